# baseline (speedup 1.0000x reference)
import functools

import jax
import jax.numpy as jnp
from jax import lax
from jax.experimental import pallas as pl
from jax.experimental.pallas import tpu as pltpu
from jax.experimental.pallas import tpu_sc as plsc

_K = 500
_KPAD = 512
_PAIRS = 4
_SLOTS = _PAIRS * _KPAD
_CHUNK = 128
_NCHUNK = _SLOTS // _CHUNK
_VECS = _SLOTS // 16
_ROW = 128 * 128


def _sc_body(gidx_hbm, tgtf_hbm, table_hbm, out_hbm,
             idx_v, tgt_v, vals_v, out_v, sem):
    cid = lax.axis_index("c")
    sid = lax.axis_index("s")

    @pl.when((cid == 0) & (sid == 0))
    def _work():
        pltpu.sync_copy(gidx_hbm, idx_v)
        pltpu.sync_copy(tgtf_hbm, tgt_v)
        copies = []
        for c in range(_NCHUNK):
            copies.append(pltpu.async_copy(
                table_hbm.at[idx_v.at[pl.ds(c * _CHUNK, _CHUNK)]],
                vals_v.at[pl.ds(c * _CHUNK, _CHUNK)], sem))
        for c in copies:
            c.wait()
        lane = lax.iota(jnp.int32, 16)
        acc = jnp.zeros((16,), jnp.float32)
        for i in range(_VECS):
            v = vals_v[pl.ds(i * 16, 16)]
            t = tgt_v[pl.ds(i * 16, 16)]
            s = i * 16 + lane
            valid = lax.rem(s, _KPAD) < _K
            acc = acc + jnp.where(valid, jnp.abs(v - t), 0.0)
        total = jnp.float32(0.0)
        for l in range(16):
            total = total + acc[l]
        loss = total * jnp.float32(1.0 / (_K + 0.0001))
        out_v[...] = jnp.where(lane == 0, loss, 0.0)
        pltpu.sync_copy(out_v, out_hbm)


@jax.jit
def kernel(out_vector, target_vector, tgt_indexes):
    b, c, h, w = out_vector.shape
    table = out_vector.reshape(b * c * h * w)

    ind = jnp.squeeze(tgt_indexes, axis=1)
    ind_p = jnp.pad(ind, ((0, 0), (0, _KPAD - _K)))
    idx_per_p = jnp.tile(ind_p, (b, 1))
    offs = (jnp.arange(_PAIRS, dtype=jnp.int32) * _ROW)[:, None]
    gidx = (idx_per_p + offs).reshape(_SLOTS).astype(jnp.int32)

    tgt = jnp.transpose(jnp.squeeze(target_vector, axis=1), (0, 2, 1))
    tgt_p = jnp.pad(tgt, ((0, 0), (0, 0), (0, _KPAD - _K)))
    tgtf = tgt_p.reshape(_SLOTS)

    run = functools.partial(
        pl.kernel,
        mesh=plsc.VectorSubcoreMesh(core_axis_name="c", subcore_axis_name="s"),
        out_type=jax.ShapeDtypeStruct((16,), jnp.float32),
        scratch_types=[
            pltpu.VMEM((_SLOTS,), jnp.int32),
            pltpu.VMEM((_SLOTS,), jnp.float32),
            pltpu.VMEM((_SLOTS,), jnp.float32),
            pltpu.VMEM((16,), jnp.float32),
            pltpu.SemaphoreType.DMA,
        ],
    )(_sc_body)
    out = run(gidx, tgtf, table)
    return out[0]


# speedup vs baseline: 1.0550x; 1.0550x over previous
import functools

import jax
import jax.numpy as jnp
from jax import lax
from jax.experimental import pallas as pl
from jax.experimental.pallas import tpu as pltpu
from jax.experimental.pallas import tpu_sc as plsc

_K = 500
_SLOTS = 4 * _K
_ROW = 128 * 128
_CHUNKS = [(c * 128, 128) for c in range(15)] + [(1920, 80)]


def _sc_body(gidx_hbm, tgtf_hbm, table_hbm, out_hbm,
             idx_v, tgt_v, vals_v, out_v, sem):
    cid = lax.axis_index("c")
    sid = lax.axis_index("s")

    @pl.when((cid == 0) & (sid == 0))
    def _work():
        pltpu.sync_copy(gidx_hbm, idx_v)
        pltpu.sync_copy(tgtf_hbm, tgt_v)
        copies = [
            pltpu.async_copy(table_hbm.at[idx_v.at[pl.ds(base, n)]],
                             vals_v.at[pl.ds(base, n)], sem)
            for base, n in _CHUNKS
        ]
        acc = jnp.zeros((16,), jnp.float32)
        for (base, n), cp in zip(_CHUNKS, copies):
            cp.wait()
            for i in range(n // 16):
                v = vals_v[pl.ds(base + i * 16, 16)]
                t = tgt_v[pl.ds(base + i * 16, 16)]
                acc = acc + jnp.abs(v - t)
        total = jnp.float32(0.0)
        for l in range(16):
            total = total + acc[l]
        loss = total * jnp.float32(1.0 / (_K + 0.0001))
        lane = lax.iota(jnp.int32, 16)
        out_v[...] = jnp.where(lane == 0, loss, 0.0)
        pltpu.sync_copy(out_v, out_hbm)


@jax.jit
def kernel(out_vector, target_vector, tgt_indexes):
    b, c, h, w = out_vector.shape
    table = out_vector.reshape(b * c * h * w)

    ind = jnp.squeeze(tgt_indexes, axis=1)
    ind_jd = jnp.transpose(ind, (1, 0)).reshape(-1)
    pair_off = (jnp.tile(jnp.arange(2, dtype=jnp.int32), (2 * _K,))
                + jnp.repeat(jnp.arange(2, dtype=jnp.int32) * 2, 2 * _K))
    gidx = (jnp.tile(ind_jd, (2,)) + pair_off * _ROW).astype(jnp.int32)

    tgtf = target_vector.reshape(_SLOTS)

    run = functools.partial(
        pl.kernel,
        mesh=plsc.VectorSubcoreMesh(core_axis_name="c", subcore_axis_name="s",
                                    num_cores=1),
        out_type=jax.ShapeDtypeStruct((16,), jnp.float32),
        scratch_types=[
            pltpu.VMEM((_SLOTS,), jnp.int32),
            pltpu.VMEM((_SLOTS,), jnp.float32),
            pltpu.VMEM((_SLOTS,), jnp.float32),
            pltpu.VMEM((16,), jnp.float32),
            pltpu.SemaphoreType.DMA,
        ],
    )(_sc_body)
    out = run(gidx, tgtf, table)
    return out[0]


# speedup vs baseline: 1.1490x; 1.0890x over previous
import functools

import jax
import jax.numpy as jnp
from jax import lax
from jax.experimental import pallas as pl
from jax.experimental.pallas import tpu as pltpu
from jax.experimental.pallas import tpu_sc as plsc

_K = 500
_SLOTS = 4 * _K
_PAD = 2048
_PER_W = _PAD // 16
_VECS = _PER_W // 16
_ROW = 128 * 128


def _sc_body(gidx_hbm, tgtf_hbm, table_hbm, out_hbm, stage_hbm,
             idx_v, tgt_v, vals_v, acc_v, out_v, red_v, sem):
    sid = lax.axis_index("s")
    base = sid * _PER_W

    pltpu.sync_copy(gidx_hbm.at[pl.ds(base, _PER_W)], idx_v)
    pltpu.sync_copy(tgtf_hbm.at[pl.ds(base, _PER_W)], tgt_v)
    pltpu.async_copy(table_hbm.at[idx_v], vals_v, sem).wait()

    lane = lax.iota(jnp.int32, 16)
    acc = jnp.zeros((16,), jnp.float32)
    for i in range(_VECS):
        v = vals_v[pl.ds(i * 16, 16)]
        t = tgt_v[pl.ds(i * 16, 16)]
        s = base + i * 16 + lane
        acc = acc + jnp.where(s < _SLOTS, jnp.abs(v - t), 0.0)
    acc_v[...] = acc
    pltpu.sync_copy(acc_v, stage_hbm.at[sid])

    plsc.subcore_barrier()

    @pl.when(sid == 0)
    def _reduce():
        pltpu.sync_copy(stage_hbm, red_v)
        tot = jnp.zeros((16,), jnp.float32)
        for i in range(16):
            tot = tot + red_v[i]
        total = jnp.float32(0.0)
        for l in range(16):
            total = total + tot[l]
        loss = total * jnp.float32(1.0 / (_K + 0.0001))
        out_v[...] = jnp.where(lane == 0, loss, 0.0)
        pltpu.sync_copy(out_v, out_hbm)


@jax.jit
def kernel(out_vector, target_vector, tgt_indexes):
    b, c, h, w = out_vector.shape
    table = out_vector.reshape(b * c * h * w)

    ind = jnp.squeeze(tgt_indexes, axis=1)
    ind_jd = jnp.transpose(ind, (1, 0)).reshape(-1)
    pair_off = (jnp.tile(jnp.arange(2, dtype=jnp.int32), (2 * _K,))
                + jnp.repeat(jnp.arange(2, dtype=jnp.int32) * 2, 2 * _K))
    gidx = (jnp.tile(ind_jd, (2,)) + pair_off * _ROW).astype(jnp.int32)
    gidx = jnp.pad(gidx, (0, _PAD - _SLOTS))

    tgtf = jnp.pad(target_vector.reshape(_SLOTS), (0, _PAD - _SLOTS))

    run = functools.partial(
        pl.kernel,
        mesh=plsc.VectorSubcoreMesh(core_axis_name="c", subcore_axis_name="s",
                                    num_cores=1),
        out_type=(jax.ShapeDtypeStruct((16,), jnp.float32),
                  jax.ShapeDtypeStruct((16, 16), jnp.float32)),
        scratch_types=[
            pltpu.VMEM((_PER_W,), jnp.int32),
            pltpu.VMEM((_PER_W,), jnp.float32),
            pltpu.VMEM((_PER_W,), jnp.float32),
            pltpu.VMEM((16,), jnp.float32),
            pltpu.VMEM((16,), jnp.float32),
            pltpu.VMEM((16, 16), jnp.float32),
            pltpu.SemaphoreType.DMA,
        ],
    )(_sc_body)
    out, _ = run(gidx, tgtf, table)
    return out[0]
